# Initial kernel scaffold; baseline (speedup 1.0000x reference)
import jax
import jax.numpy as jnp
from jax.experimental import pallas as pl


def kernel(boxes, scores):
    raise NotImplementedError("write your pallas kernel here")


# speedup vs baseline: 37.2139x; 37.2139x over previous
import jax
import jax.numpy as jnp
from jax.experimental import pallas as pl
from jax.experimental.pallas import tpu as pltpu

_N = 20000
_LANES = 128
_ROWS = 160
_PAD = _ROWS * _LANES
_MAX_DET = 300
_IOU_THR = 0.45
_CONF_THR = 0.25


def _nms_kernel(x1_ref, y1_ref, x2_ref, y2_ref, s_ref, out_ref, area_ref):
    x1 = x1_ref[...]
    y1 = y1_ref[...]
    x2 = x2_ref[...]
    y2 = y2_ref[...]
    area_ref[...] = (x2 - x1) * (y2 - y1)

    s0 = s_ref[...]
    s0 = jnp.where(s0 >= _CONF_THR, s0, 0.0)

    row_i = jax.lax.broadcasted_iota(jnp.int32, (_ROWS, _LANES), 0)
    col_i = jax.lax.broadcasted_iota(jnp.int32, (_ROWS, _LANES), 1)
    flat_i = row_i * _LANES + col_i
    lane_i = jax.lax.broadcasted_iota(jnp.int32, (1, _LANES), 1)

    m0 = jnp.max(s0)
    idx0 = jnp.min(jnp.where(s0 == m0, flat_i, _PAD))

    def body(i, carry):
        s, m, idx = carry
        r = idx // _LANES
        c = idx % _LANES
        cmask = lane_i == c
        bx1 = jnp.sum(jnp.where(cmask, x1_ref[pl.ds(r, 1), :], 0.0))
        by1 = jnp.sum(jnp.where(cmask, y1_ref[pl.ds(r, 1), :], 0.0))
        bx2 = jnp.sum(jnp.where(cmask, x2_ref[pl.ds(r, 1), :], 0.0))
        by2 = jnp.sum(jnp.where(cmask, y2_ref[pl.ds(r, 1), :], 0.0))
        barea = (bx2 - bx1) * (by2 - by1)

        xx1 = jnp.maximum(bx1, x1_ref[...])
        yy1 = jnp.maximum(by1, y1_ref[...])
        xx2 = jnp.minimum(bx2, x2_ref[...])
        yy2 = jnp.minimum(by2, y2_ref[...])
        inter = jnp.maximum(xx2 - xx1, 0.0) * jnp.maximum(yy2 - yy1, 0.0)
        iou = inter / (barea + area_ref[...] - inter + 1e-9)
        s_new = jnp.where(iou > _IOU_THR, 0.0, s)

        m_new = jnp.max(s_new)
        idx_new = jnp.min(jnp.where(s_new == m_new, flat_i, _PAD))

        valid = m > 0.0
        entry = (jnp.where(lane_i == 0, bx1, 0.0)
                 + jnp.where(lane_i == 1, by1, 0.0)
                 + jnp.where(lane_i == 2, bx2, 0.0)
                 + jnp.where(lane_i == 3, by2, 0.0)
                 + jnp.where(lane_i == 4, m, 0.0))
        out_ref[pl.ds(i, 1), :] = jnp.where(valid, entry, 0.0)
        return (s_new, m_new, idx_new)

    jax.lax.fori_loop(0, _MAX_DET, body, (s0, m0, idx0), unroll=False)


def kernel(boxes, scores):
    pb = jnp.pad(boxes, ((0, _PAD - _N), (0, 0)))
    x1 = pb[:, 0].reshape(_ROWS, _LANES)
    y1 = pb[:, 1].reshape(_ROWS, _LANES)
    x2 = pb[:, 2].reshape(_ROWS, _LANES)
    y2 = pb[:, 3].reshape(_ROWS, _LANES)
    s = jnp.pad(scores, (0, _PAD - _N)).reshape(_ROWS, _LANES)

    out = pl.pallas_call(
        _nms_kernel,
        out_shape=jax.ShapeDtypeStruct((_MAX_DET, _LANES), jnp.float32),
        scratch_shapes=[pltpu.VMEM((_ROWS, _LANES), jnp.float32)],
    )(x1, y1, x2, y2, s)
    return out[:, :5]


# speedup vs baseline: 38.5929x; 1.0371x over previous
import jax
import jax.numpy as jnp
from jax.experimental import pallas as pl
from jax.experimental.pallas import tpu as pltpu

_N = 20000
_LANES = 128
_ROWS = 160
_PAD = _ROWS * _LANES
_MAX_DET = 300
_IOU_THR = 0.45
_CONF_THR = 0.25


def _nms_kernel(x1_ref, y1_ref, x2_ref, y2_ref, s_ref, out_ref, area_ref):
    x1 = x1_ref[...]
    y1 = y1_ref[...]
    x2 = x2_ref[...]
    y2 = y2_ref[...]
    area_ref[...] = (x2 - x1) * (y2 - y1)

    s0 = s_ref[...]
    s0 = jnp.where(s0 >= _CONF_THR, s0, 0.0)

    row_i = jax.lax.broadcasted_iota(jnp.int32, (_ROWS, _LANES), 0)
    col_i = jax.lax.broadcasted_iota(jnp.int32, (_ROWS, _LANES), 1)
    flat_i = row_i * _LANES + col_i
    lane_i = jax.lax.broadcasted_iota(jnp.int32, (1, _LANES), 1)

    m0 = jnp.max(s0)
    idx0 = jnp.min(jnp.where(s0 == m0, flat_i, _PAD))

    def body(i, carry):
        s, m, idx = carry
        r = idx // _LANES
        c = idx % _LANES
        cmask = lane_i == c
        bx1 = jnp.sum(jnp.where(cmask, x1_ref[pl.ds(r, 1), :], 0.0))
        by1 = jnp.sum(jnp.where(cmask, y1_ref[pl.ds(r, 1), :], 0.0))
        bx2 = jnp.sum(jnp.where(cmask, x2_ref[pl.ds(r, 1), :], 0.0))
        by2 = jnp.sum(jnp.where(cmask, y2_ref[pl.ds(r, 1), :], 0.0))
        barea = (bx2 - bx1) * (by2 - by1)

        xx1 = jnp.maximum(bx1, x1_ref[...])
        yy1 = jnp.maximum(by1, y1_ref[...])
        xx2 = jnp.minimum(bx2, x2_ref[...])
        yy2 = jnp.minimum(by2, y2_ref[...])
        inter = jnp.maximum(xx2 - xx1, 0.0) * jnp.maximum(yy2 - yy1, 0.0)
        iou = inter / (barea + area_ref[...] - inter + 1e-9)
        s_new = jnp.where(iou > _IOU_THR, 0.0, s)

        colmax = jnp.max(s_new, axis=0, keepdims=True)
        colidx = jnp.min(jnp.where(s_new == colmax, flat_i, _PAD),
                         axis=0, keepdims=True)
        m_new = jnp.max(colmax)
        idx_new = jnp.min(jnp.where(colmax == m_new, colidx, _PAD))

        valid = m > 0.0
        entry = (jnp.where(lane_i == 0, bx1, 0.0)
                 + jnp.where(lane_i == 1, by1, 0.0)
                 + jnp.where(lane_i == 2, bx2, 0.0)
                 + jnp.where(lane_i == 3, by2, 0.0)
                 + jnp.where(lane_i == 4, m, 0.0))
        out_ref[pl.ds(i, 1), :] = jnp.where(valid, entry, 0.0)
        return (s_new, m_new, idx_new)

    jax.lax.fori_loop(0, _MAX_DET, body, (s0, m0, idx0), unroll=False)


def kernel(boxes, scores):
    pb = jnp.pad(boxes, ((0, _PAD - _N), (0, 0)))
    x1 = pb[:, 0].reshape(_ROWS, _LANES)
    y1 = pb[:, 1].reshape(_ROWS, _LANES)
    x2 = pb[:, 2].reshape(_ROWS, _LANES)
    y2 = pb[:, 3].reshape(_ROWS, _LANES)
    s = jnp.pad(scores, (0, _PAD - _N)).reshape(_ROWS, _LANES)

    out = pl.pallas_call(
        _nms_kernel,
        out_shape=jax.ShapeDtypeStruct((_MAX_DET, _LANES), jnp.float32),
        scratch_shapes=[pltpu.VMEM((_ROWS, _LANES), jnp.float32)],
    )(x1, y1, x2, y2, s)
    return out[:, :5]
